# baseline (speedup 1.0000x reference)
import functools
import jax
import jax.numpy as jnp
from jax import lax
from jax.experimental import pallas as pl
from jax.experimental.pallas import tpu as pltpu
from jax.experimental.pallas import tpu_sc as plsc

N_KEEP = 48 * 256
NC, NS = 2, 16
NW = NC * NS
CHUNK = 1
NBUF = 8

_mesh = plsc.VectorSubcoreMesh(core_axis_name="c", subcore_axis_name="s")


def _make_sc_copy(batch):
    rows_per_w = batch // NW
    nchunk = rows_per_w // CHUNK

    @functools.partial(
        pl.kernel,
        mesh=_mesh,
        out_type=jax.ShapeDtypeStruct((batch, N_KEEP), jnp.float32),
        scratch_types=[
            pltpu.VMEM((NBUF, CHUNK, N_KEEP), jnp.float32),
            pltpu.SemaphoreType.DMA,
            pltpu.SemaphoreType.DMA,
        ],
    )
    def _sc_copy(in_hbm, out_hbm, buf, sem_in, sem_out):
        wid = lax.axis_index("s") * NC + lax.axis_index("c")
        base = wid * rows_per_w

        def in_copy(g, slot):
            r0 = base + g * CHUNK
            return pltpu.make_async_copy(
                in_hbm.at[pl.ds(r0, CHUNK), pl.ds(0, N_KEEP)],
                buf.at[slot], sem_in)

        def out_copy(g, slot):
            r0 = base + g * CHUNK
            return pltpu.make_async_copy(
                buf.at[slot], out_hbm.at[pl.ds(r0, CHUNK)], sem_out)

        for g in range(NBUF):
            in_copy(g, g).start()
        for g in range(nchunk):
            slot = g % NBUF
            in_copy(g, slot).wait()
            oc = out_copy(g, slot)
            oc.start()
            oc.wait()
            if g + NBUF < nchunk:
                in_copy(g + NBUF, slot).start()

    return _sc_copy


def kernel(inputs):
    return _make_sc_copy(inputs.shape[0])(inputs)


# speedup vs baseline: 1.0383x; 1.0383x over previous
import functools
import jax
import jax.numpy as jnp
from jax import lax
from jax.experimental import pallas as pl
from jax.experimental.pallas import tpu as pltpu
from jax.experimental.pallas import tpu_sc as plsc

N_KEEP = 48 * 256
NC, NS = 2, 16
NW = NC * NS
CHUNK = 4
NBUF = 2

_mesh = plsc.VectorSubcoreMesh(core_axis_name="c", subcore_axis_name="s")


def _make_sc_copy(batch):
    rows_per_w = batch // NW
    nchunk = rows_per_w // CHUNK

    @functools.partial(
        pl.kernel,
        mesh=_mesh,
        out_type=jax.ShapeDtypeStruct((batch, N_KEEP), jnp.float32),
        scratch_types=[
            pltpu.VMEM((NBUF, CHUNK, N_KEEP), jnp.float32),
            pltpu.SemaphoreType.DMA,
            pltpu.SemaphoreType.DMA,
        ],
    )
    def _sc_copy(in_hbm, out_hbm, buf, sem_in, sem_out):
        wid = lax.axis_index("s") * NC + lax.axis_index("c")
        base = wid * rows_per_w

        def in_copy(g, slot):
            r0 = base + g * CHUNK
            return pltpu.make_async_copy(
                in_hbm.at[pl.ds(r0, CHUNK), pl.ds(0, N_KEEP)],
                buf.at[slot], sem_in)

        def out_copy(g, slot):
            r0 = base + g * CHUNK
            return pltpu.make_async_copy(
                buf.at[slot], out_hbm.at[pl.ds(r0, CHUNK)], sem_out)

        for g in range(NBUF):
            in_copy(g, g).start()
        for g in range(nchunk):
            slot = g % NBUF
            in_copy(g, slot).wait()
            oc = out_copy(g, slot)
            oc.start()
            oc.wait()
            if g + NBUF < nchunk:
                in_copy(g + NBUF, slot).start()

    return _sc_copy


def kernel(inputs):
    return _make_sc_copy(inputs.shape[0])(inputs)
